# baseline (speedup 1.0000x reference)
import functools

import jax
import jax.numpy as jnp
from jax import lax
from jax.experimental import pallas as pl
from jax.experimental.pallas import tpu as pltpu
from jax.experimental.pallas import tpu_sc as plsc

N = 50000
E = 800000
G = 50
A = 1000
H = 2
C = 16
NG = 50
HC = H * C

NC = 2
NS = 16
NW = NC * NS

CH = 128
CHE = 24
EPT = 25344
E_pad = NW * EPT
CPT_G = EPT // CH
CPT_E = EPT // CHE
CPT_S = E_pad // NS // CH
NP = 50176
STR = NP // NS
PW = 40

COEFF = -0.5 / float(5.0 / (NG - 1)) ** 2
SLOPE = 0.2

BLKE = 512
BLKN = 1000

_mesh = lambda: plsc.VectorSubcoreMesh(
    core_axis_name="c", subcore_axis_name="s", num_cores=NC, num_subcores=NS)

_f32 = jnp.float32
_HIGH = lax.Precision.HIGHEST


def _leaky(x):
  return jnp.maximum(x, x * SLOPE)


def _sc_gather_pos(pos4, srcp, dsts, interpret=False):
  @functools.partial(
      pl.kernel,
      out_type=(jax.ShapeDtypeStruct((E_pad, 16), _f32),
                jax.ShapeDtypeStruct((E_pad, 16), _f32)),
      mesh=_mesh(),
      compiler_params=pltpu.CompilerParams(use_tc_tiling_on_sc=False, needs_layout_passes=False),
      scratch_types=[
          pltpu.VMEM((CH,), jnp.int32), pltpu.VMEM((CH,), jnp.int32),
          pltpu.VMEM((CH, 16), _f32), pltpu.VMEM((CH, 16), _f32),
          pltpu.SemaphoreType.DMA, pltpu.SemaphoreType.DMA,
      ],
      interpret=interpret)
  def k(pos4_h, srcp_h, dsts_h, ps_h, pd_h, sidx, didx, sbuf, dbuf, sem1, sem2):
    wid = lax.axis_index("s") * NC + lax.axis_index("c")
    tbase = wid * EPT

    def chunk(i, carry):
      base = tbase + i * CH
      pltpu.sync_copy(srcp_h.at[pl.ds(base, CH)], sidx)
      pltpu.sync_copy(dsts_h.at[pl.ds(base, CH)], didx)
      c1 = pltpu.async_copy(pos4_h.at[sidx], sbuf, sem1)
      c2 = pltpu.async_copy(pos4_h.at[didx], dbuf, sem2)
      c1.wait()
      c2.wait()
      pltpu.sync_copy(sbuf, ps_h.at[pl.ds(base, CH)])
      pltpu.sync_copy(dbuf, pd_h.at[pl.ds(base, CH)])
      return carry

    lax.fori_loop(0, CPT_G, chunk, 0)

  return k(pos4, srcp, dsts)


def _sc_scatter_pre(eprojs, dsts, zstripe, interpret=False):
  @functools.partial(
      pl.kernel,
      out_type=jax.ShapeDtypeStruct((2, NP, PW), _f32),
      mesh=_mesh(),
      compiler_params=pltpu.CompilerParams(use_tc_tiling_on_sc=False, needs_layout_passes=False),
      scratch_types=[
          pltpu.VMEM((CH,), jnp.int32),
          pltpu.VMEM((CH, PW), _f32),
          pltpu.VMEM_SHARED((NP, PW), _f32),
      ],
      interpret=interpret)
  def k(eprojs_h, dsts_h, z_h, acc_h, didx, ebuf, acc):
    cid = lax.axis_index("c")
    sid = lax.axis_index("s")
    pltpu.sync_copy(z_h, acc.at[pl.ds(sid * STR, STR)])
    plsc.subcore_barrier()
    tbase = sid * (CPT_S * CH)

    def chunk(i, carry):
      base = tbase + i * CH
      pltpu.sync_copy(dsts_h.at[pl.ds(base, CH)], didx)

      @pl.when(cid == 0)
      def _():
        pltpu.sync_copy(eprojs_h.at[0, pl.ds(base, CH)], ebuf)

      @pl.when(cid == 1)
      def _():
        pltpu.sync_copy(eprojs_h.at[1, pl.ds(base, CH)], ebuf)

      pltpu.sync_copy(ebuf, acc.at[didx], add=True)
      return carry

    lax.fori_loop(0, CPT_S, chunk, 0)
    plsc.subcore_barrier()

    @pl.when(cid == 0)
    def _():
      pltpu.sync_copy(acc.at[pl.ds(sid * STR, STR)],
                      acc_h.at[0, pl.ds(sid * STR, STR)])

    @pl.when(cid == 1)
    def _():
      pltpu.sync_copy(acc.at[pl.ds(sid * STR, STR)],
                      acc_h.at[1, pl.ds(sid * STR, STR)])

  return k(eprojs, dsts, zstripe)


def _sc_edge_pass(srcp, dsts, eprojs, xl, xrp, att, zstripe, slab,
                  interpret=False):
  @functools.partial(
      pl.kernel,
      out_type=jax.ShapeDtypeStruct((2, NP, PW), _f32),
      mesh=_mesh(),
      compiler_params=pltpu.CompilerParams(use_tc_tiling_on_sc=False, needs_layout_passes=False),
      scratch_types=[
          pltpu.VMEM((2, CHE), jnp.int32), pltpu.VMEM((2, CHE), jnp.int32),
          pltpu.VMEM((2, CHE, HC), _f32), pltpu.VMEM((2, CHE, HC), _f32),
          pltpu.VMEM((2, CHE, PW), _f32),
          pltpu.VMEM((2, 16), _f32),
          pltpu.VMEM_SHARED((NP, PW), _f32),
          pltpu.SemaphoreType.DMA, pltpu.SemaphoreType.DMA,
          pltpu.SemaphoreType.DMA, pltpu.SemaphoreType.DMA,
      ],
      interpret=interpret)
  def k(srcp_h, dsts_h, eprojs_h, xl_h, xr_h, att_h, z_h, acc_h,
        sidx, didx, xlb, xrb, pay, attv, acc, s1a, s1b, s2a, s2b):
    cid = lax.axis_index("c")
    sid = lax.axis_index("s")
    pltpu.sync_copy(z_h, acc.at[pl.ds(sid * STR, STR)])
    pltpu.sync_copy(att_h, attv)
    plsc.subcore_barrier()
    wid = cid * NS + sid
    tbase = wid * EPT
    gsem = (s1a, s1b)
    rsem = (s2a, s2b)
    att0 = attv[0]
    att1 = attv[1]
    lane = lax.broadcasted_iota(jnp.int32, (16,), 0)

    def pair(g, carry):
      cps = []
      for b in range(2):
        i = g * 2 + b
        base = tbase + i * CHE
        pltpu.sync_copy(srcp_h.at[pl.ds(base, CHE)], sidx.at[b])
        pltpu.sync_copy(dsts_h.at[pl.ds(base, CHE)], didx.at[b])
        cps.append((pltpu.async_copy(xl_h.at[sidx.at[b]], xlb.at[b], gsem[b]),
                    pltpu.async_copy(xr_h.at[didx.at[b]], xrb.at[b], rsem[b])))
      for b in range(2):
        i = g * 2 + b
        base = tbase + i * CHE
        pltpu.sync_copy(eprojs_h.at[slab, pl.ds(base, CHE)], pay.at[b])
        cps[b][0].wait()
        cps[b][1].wait()

        def edge4(q, carry2):
          for u in range(4):
            e = q * 4 + u
            xl0 = xlb[b, e, pl.ds(0, 16)]
            xl1v = xlb[b, e, pl.ds(16, 16)]
            m0 = xl0 + xrb[b, e, pl.ds(0, 16)] + pay[b, e, pl.ds(0, 16)]
            m1 = xl1v + xrb[b, e, pl.ds(16, 16)] + pay[b, e, pl.ds(16, 16)]
            s0 = jnp.sum(_leaky(m0) * att0)
            s1 = jnp.sum(_leaky(m1) * att1)
            p0 = jnp.exp(lax.broadcast_in_dim(s0, (16,), ()))
            p1 = jnp.exp(lax.broadcast_in_dim(s1, (16,), ()))
            pay[b, e, pl.ds(24, 16)] = jnp.where(lane == 8, p0, p1)
            pay[b, e, pl.ds(0, 16)] = p0 * xl0
            pay[b, e, pl.ds(16, 16)] = p1 * xl1v
          return carry2

        lax.fori_loop(0, CHE // 4, edge4, 0)
        pltpu.sync_copy(pay.at[b], acc.at[didx.at[b]], add=True)
      return carry

    lax.fori_loop(0, CPT_E // 2, pair, 0)
    plsc.subcore_barrier()

    @pl.when(cid == 0)
    def _():
      pltpu.sync_copy(acc.at[pl.ds(sid * STR, STR)],
                      acc_h.at[0, pl.ds(sid * STR, STR)])

    @pl.when(cid == 1)
    def _():
      pltpu.sync_copy(acc.at[pl.ds(sid * STR, STR)],
                      acc_h.at[1, pl.ds(sid * STR, STR)])

  return k(srcp, dsts, eprojs, xl, xrp, att, zstripe)


def _tc_edge_dense(ps4, pd4, We1, We2, interpret=False):
  def body(ps_ref, pd_ref, w1_ref, w2_ref, out_ref):
    df = ps_ref[...] - pd_ref[...]
    d2 = jnp.sum(df * df, axis=1, keepdims=True)
    d = jnp.sqrt(d2 + 1e-12)
    offs = lax.broadcasted_iota(jnp.int32, (1, NG), 1).astype(_f32) * (
        5.0 / (NG - 1))
    ea = jnp.exp(COEFF * (d - offs) ** 2)
    e1 = jnp.dot(ea, w1_ref[...], precision=_HIGH)
    e2 = jnp.dot(ea, w2_ref[...], precision=_HIGH)
    out_ref[0, :, 0:HC] = e1
    out_ref[0, :, HC:HC + 1] = jnp.ones((BLKE, 1), _f32)
    out_ref[0, :, HC + 1:PW] = jnp.zeros((BLKE, PW - HC - 1), _f32)
    out_ref[1, :, 0:HC] = e2
    out_ref[1, :, HC:PW] = jnp.zeros((BLKE, PW - HC), _f32)

  grid = E_pad // BLKE
  return pl.pallas_call(
      body,
      grid=(grid,),
      in_specs=[
          pl.BlockSpec((BLKE, 16), lambda i: (i, 0)),
          pl.BlockSpec((BLKE, 16), lambda i: (i, 0)),
          pl.BlockSpec((NG, HC), lambda i: (0, 0)),
          pl.BlockSpec((NG, HC), lambda i: (0, 0)),
      ],
      out_specs=pl.BlockSpec((2, BLKE, PW), lambda i: (0, i, 0)),
      out_shape=jax.ShapeDtypeStruct((2, E_pad, PW), _f32),
      interpret=interpret,
  )(ps4, pd4, We1, We2)


def _tc_node_pre(pos, accS, W_init, b_init, Wl1, bl1, Wr1, br1, att1,
                 interpret=False):
  def body(pos_ref, a_ref, wi_ref, bi_ref, wl_ref, bl_ref, wr_ref, br_ref,
           at_ref, xl_o, xr_o, pl_o, le2_o, cf_o):
    x0 = jax.nn.softplus(
        jnp.dot(pos_ref[...], wi_ref[...], precision=_HIGH) + bi_ref[...])
    a0 = a_ref[0]
    a1 = a_ref[1]
    cnt = a0[:, HC:HC + 1]
    ccl = jnp.maximum(cnt, 1.0)
    le1 = a0[:, 0:HC] / ccl
    le2 = a1[:, 0:HC] / ccl
    cf_o[...] = cnt + 1.0
    le2_o[...] = le2
    xl = jnp.dot(x0, wl_ref[...], precision=_HIGH) + bl_ref[...]
    xr = jnp.dot(x0, wr_ref[...], precision=_HIGH) + br_ref[...]
    xl_o[...] = xl
    xr_o[...] = xr
    ml = _leaky(xl + xr + le1).reshape(BLKN, H, C)
    al = jnp.sum(ml * at_ref[...], axis=-1)
    pl_o[...] = jnp.exp(al)

  grid = N // BLKN
  return pl.pallas_call(
      body,
      grid=(grid,),
      in_specs=[
          pl.BlockSpec((BLKN, 3), lambda i: (i, 0)),
          pl.BlockSpec((2, BLKN, PW), lambda i: (0, i, 0)),
          pl.BlockSpec((3, C), lambda i: (0, 0)),
          pl.BlockSpec((1, C), lambda i: (0, 0)),
          pl.BlockSpec((C, HC), lambda i: (0, 0)),
          pl.BlockSpec((1, HC), lambda i: (0, 0)),
          pl.BlockSpec((C, HC), lambda i: (0, 0)),
          pl.BlockSpec((1, HC), lambda i: (0, 0)),
          pl.BlockSpec((1, H, C), lambda i: (0, 0, 0)),
      ],
      out_specs=[
          pl.BlockSpec((BLKN, HC), lambda i: (i, 0)),
          pl.BlockSpec((BLKN, HC), lambda i: (i, 0)),
          pl.BlockSpec((BLKN, H), lambda i: (i, 0)),
          pl.BlockSpec((BLKN, HC), lambda i: (i, 0)),
          pl.BlockSpec((BLKN, 1), lambda i: (i, 0)),
      ],
      out_shape=[
          jax.ShapeDtypeStruct((N, HC), _f32),
          jax.ShapeDtypeStruct((N, HC), _f32),
          jax.ShapeDtypeStruct((N, H), _f32),
          jax.ShapeDtypeStruct((N, HC), _f32),
          jax.ShapeDtypeStruct((N, 1), _f32),
      ],
      interpret=interpret,
  )(pos, accS, W_init, b_init, Wl1, bl1, Wr1, br1, att1)


def _combine(accE, xl, ploop, cntf, bias):
  asum = accE[0] + accE[1]
  num = (asum[:, 0:HC].reshape(BLKN, H, C)
         + ploop[:, :, None] * xl.reshape(BLKN, H, C))
  den = asum[:, HC:HC + H] + ploop
  out = jnp.sum(num / den[:, :, None], axis=1) * (0.5 / cntf) + bias
  return jax.nn.softplus(out)


def _tc_combine_mid(accE, xl, ploop, cntf, bias, le_n, Wl_n, bl_n, Wr_n, br_n,
                    att_n, interpret=False):
  def body(a_ref, xl_ref, pl_ref, cf_ref, b_ref, le_ref, wl_ref, bl_ref,
           wr_ref, br_ref, at_ref, xl_o, xr_o, pl_o):
    xn = _combine(a_ref[...], xl_ref[...], pl_ref[...], cf_ref[...], b_ref[...])
    xln = jnp.dot(xn, wl_ref[...], precision=_HIGH) + bl_ref[...]
    xrn = jnp.dot(xn, wr_ref[...], precision=_HIGH) + br_ref[...]
    xl_o[...] = xln
    xr_o[...] = xrn
    ml = _leaky(xln + xrn + le_ref[...]).reshape(BLKN, H, C)
    pl_o[...] = jnp.exp(jnp.sum(ml * at_ref[...], axis=-1))

  grid = N // BLKN
  return pl.pallas_call(
      body,
      grid=(grid,),
      in_specs=[
          pl.BlockSpec((2, BLKN, PW), lambda i: (0, i, 0)),
          pl.BlockSpec((BLKN, HC), lambda i: (i, 0)),
          pl.BlockSpec((BLKN, H), lambda i: (i, 0)),
          pl.BlockSpec((BLKN, 1), lambda i: (i, 0)),
          pl.BlockSpec((1, C), lambda i: (0, 0)),
          pl.BlockSpec((BLKN, HC), lambda i: (i, 0)),
          pl.BlockSpec((C, HC), lambda i: (0, 0)),
          pl.BlockSpec((1, HC), lambda i: (0, 0)),
          pl.BlockSpec((C, HC), lambda i: (0, 0)),
          pl.BlockSpec((1, HC), lambda i: (0, 0)),
          pl.BlockSpec((1, H, C), lambda i: (0, 0, 0)),
      ],
      out_specs=[
          pl.BlockSpec((BLKN, HC), lambda i: (i, 0)),
          pl.BlockSpec((BLKN, HC), lambda i: (i, 0)),
          pl.BlockSpec((BLKN, H), lambda i: (i, 0)),
      ],
      out_shape=[
          jax.ShapeDtypeStruct((N, HC), _f32),
          jax.ShapeDtypeStruct((N, HC), _f32),
          jax.ShapeDtypeStruct((N, H), _f32),
      ],
      interpret=interpret,
  )(accE, xl, ploop, cntf, bias, le_n, Wl_n, bl_n, Wr_n, br_n, att_n)


def _tc_combine_final(accE, xl, ploop, cntf, bias, W_p1, b_p1, W_p2, b_p2,
                      sig, interpret=False):
  def body(a_ref, xl_ref, pl_ref, cf_ref, b_ref, w1_ref, b1_ref, w2_ref,
           b2_ref, sg_ref, out_o):
    xn = _combine(a_ref[...], xl_ref[...], pl_ref[...], cf_ref[...], b_ref[...])
    y = jax.nn.softplus(
        jnp.dot(xn, w1_ref[...], precision=_HIGH) + b1_ref[...])
    sc = jnp.dot(y, w2_ref[...], precision=_HIGH) + b2_ref[...]
    out_o[...] = sc / sg_ref[...]

  grid = N // BLKN
  return pl.pallas_call(
      body,
      grid=(grid,),
      in_specs=[
          pl.BlockSpec((2, BLKN, PW), lambda i: (0, i, 0)),
          pl.BlockSpec((BLKN, HC), lambda i: (i, 0)),
          pl.BlockSpec((BLKN, H), lambda i: (i, 0)),
          pl.BlockSpec((BLKN, 1), lambda i: (i, 0)),
          pl.BlockSpec((1, C), lambda i: (0, 0)),
          pl.BlockSpec((C, C), lambda i: (0, 0)),
          pl.BlockSpec((1, C), lambda i: (0, 0)),
          pl.BlockSpec((C, 3), lambda i: (0, 0)),
          pl.BlockSpec((1, 3), lambda i: (0, 0)),
          pl.BlockSpec((BLKN, 1), lambda i: (i, 0)),
      ],
      out_specs=pl.BlockSpec((BLKN, 3), lambda i: (i, 0)),
      out_shape=jax.ShapeDtypeStruct((N, 3), _f32),
      interpret=interpret,
  )(accE, xl, ploop, cntf, bias, W_p1, b_p1, W_p2, b_p2, sig)


def kernel(pos, edge_index, sigmas, W_init, b_init, Wl1, bl1, Wr1, br1, We1,
           att1, bias1, Wl2, bl2, Wr2, br2, We2, att2, bias2, W_p1, b_p1,
           W_p2, b_p2):
  src = edge_index[0]
  dst = edge_index[1]
  padn = E_pad - E + CHE
  srcp = jnp.concatenate([src, jnp.zeros((padn,), jnp.int32)])
  dsts = jnp.concatenate([dst, jnp.full((padn,), N, jnp.int32)])
  pos4 = jnp.pad(pos, ((0, 16), (0, 13)))
  zstripe = jnp.zeros((STR, PW), _f32)
  sig = sigmas.reshape(N, 1)
  b_init2 = b_init.reshape(1, C)
  bl1r, br1r = bl1.reshape(1, HC), br1.reshape(1, HC)
  bl2r, br2r = bl2.reshape(1, HC), br2.reshape(1, HC)
  bias1r, bias2r = bias1.reshape(1, C), bias2.reshape(1, C)
  att1r = att1.reshape(H, C)
  att2r = att2.reshape(H, C)

  ps4, pd4 = _sc_gather_pos(pos4, srcp, dsts)
  eprojs = _tc_edge_dense(ps4, pd4, We1, We2)
  accS = _sc_scatter_pre(eprojs, dsts, zstripe)
  xl, xr, ploop, le2, cntf = _tc_node_pre(
      pos, accS, W_init, b_init2, Wl1, bl1r, Wr1, br1r, att1)

  accE = _sc_edge_pass(srcp, dsts, eprojs, xl,
                       jnp.pad(xr, ((0, 16), (0, 0))), att1r, zstripe, 0)
  xl, xr, ploop2 = _tc_combine_mid(
      accE, xl, ploop, cntf, bias1r, le2, Wl2, bl2r, Wr2, br2r, att2)
  accE = _sc_edge_pass(srcp, dsts, eprojs, xl,
                       jnp.pad(xr, ((0, 16), (0, 0))), att2r, zstripe, 1)
  xl, xr, ploop3 = _tc_combine_mid(
      accE, xl, ploop2, cntf, bias2r, le2, Wl2, bl2r, Wr2, br2r, att2)
  accE = _sc_edge_pass(srcp, dsts, eprojs, xl,
                       jnp.pad(xr, ((0, 16), (0, 0))), att2r, zstripe, 1)
  scores = _tc_combine_final(
      accE, xl, ploop3, cntf, bias2r, W_p1, b_p1.reshape(1, C), W_p2,
      b_p2.reshape(1, 3), sig)
  return scores


# speedup vs baseline: 1.2364x; 1.2364x over previous
import functools

import jax
import jax.numpy as jnp
from jax import lax
from jax.experimental import pallas as pl
from jax.experimental.pallas import tpu as pltpu
from jax.experimental.pallas import tpu_sc as plsc

N = 50000
E = 800000
G = 50
A = 1000
H = 2
C = 16
NG = 50
HC = H * C

NC = 2
NS = 16
NW = NC * NS

CH = 128
CHE = 48
EPT = 25344
E_pad = NW * EPT
CPT_G = EPT // CH
CPT_E = EPT // CHE
CPT_S = E_pad // NS // CH
NP = 50176
STR = NP // NS
PW = 40

COEFF = -0.5 / float(5.0 / (NG - 1)) ** 2
SLOPE = 0.2

BLKE = 512
BLKN = 1000

_mesh = lambda: plsc.VectorSubcoreMesh(
    core_axis_name="c", subcore_axis_name="s", num_cores=NC, num_subcores=NS)

_f32 = jnp.float32
_HIGH = lax.Precision.HIGHEST


def _leaky(x):
  return jnp.maximum(x, x * SLOPE)


def _sc_gather_pos(pos4, srcp, dstg, interpret=False):
  @functools.partial(
      pl.kernel,
      out_type=(jax.ShapeDtypeStruct((E_pad, 16), _f32),
                jax.ShapeDtypeStruct((E_pad, 16), _f32)),
      mesh=_mesh(),
      compiler_params=pltpu.CompilerParams(use_tc_tiling_on_sc=False, needs_layout_passes=False),
      scratch_types=[
          pltpu.VMEM((CH,), jnp.int32), pltpu.VMEM((CH,), jnp.int32),
          pltpu.VMEM((CH, 16), _f32), pltpu.VMEM((CH, 16), _f32),
          pltpu.SemaphoreType.DMA, pltpu.SemaphoreType.DMA,
      ],
      interpret=interpret)
  def k(pos4_h, srcp_h, dstg_h, ps_h, pd_h, sidx, didx, sbuf, dbuf, sem1, sem2):
    wid = lax.axis_index("s") * NC + lax.axis_index("c")
    tbase = wid * EPT

    def chunk(i, carry):
      base = tbase + i * CH
      pltpu.sync_copy(srcp_h.at[pl.ds(base, CH)], sidx)
      pltpu.sync_copy(dstg_h.at[pl.ds(base, CH)], didx)
      c1 = pltpu.async_copy(pos4_h.at[sidx], sbuf, sem1)
      c2 = pltpu.async_copy(pos4_h.at[didx], dbuf, sem2)
      c1.wait()
      c2.wait()
      pltpu.sync_copy(sbuf, ps_h.at[pl.ds(base, CH)])
      pltpu.sync_copy(dbuf, pd_h.at[pl.ds(base, CH)])
      return carry

    lax.fori_loop(0, CPT_G, chunk, 0)

  return k(pos4, srcp, dstg)


def _sc_scatter_pre(eprojs, dsts, zstripe, interpret=False):
  @functools.partial(
      pl.kernel,
      out_type=jax.ShapeDtypeStruct((2, NP, PW), _f32),
      mesh=_mesh(),
      compiler_params=pltpu.CompilerParams(use_tc_tiling_on_sc=False, needs_layout_passes=False),
      scratch_types=[
          pltpu.VMEM((CH,), jnp.int32),
          pltpu.VMEM((CH, PW), _f32),
          pltpu.VMEM_SHARED((NP, PW), _f32),
      ],
      interpret=interpret)
  def k(eprojs_h, dsts_h, z_h, acc_h, didx, ebuf, acc):
    cid = lax.axis_index("c")
    sid = lax.axis_index("s")
    pltpu.sync_copy(z_h, acc.at[pl.ds(sid * STR, STR)])
    plsc.subcore_barrier()
    tbase = sid * (CPT_S * CH)

    def chunk(i, carry):
      base = tbase + i * CH
      pltpu.sync_copy(dsts_h.at[pl.ds(base, CH)], didx)

      @pl.when(cid == 0)
      def _():
        pltpu.sync_copy(eprojs_h.at[0, pl.ds(base, CH)], ebuf)

      @pl.when(cid == 1)
      def _():
        pltpu.sync_copy(eprojs_h.at[1, pl.ds(base, CH)], ebuf)

      pltpu.sync_copy(ebuf, acc.at[didx], add=True)
      return carry

    lax.fori_loop(0, CPT_S, chunk, 0)
    plsc.subcore_barrier()

    @pl.when(cid == 0)
    def _():
      pltpu.sync_copy(acc.at[pl.ds(sid * STR, STR)],
                      acc_h.at[0, pl.ds(sid * STR, STR)])

    @pl.when(cid == 1)
    def _():
      pltpu.sync_copy(acc.at[pl.ds(sid * STR, STR)],
                      acc_h.at[1, pl.ds(sid * STR, STR)])

  return k(eprojs, dsts, zstripe)


def _sc_edge_pass(sd, eprojs, xl, xr, att, zstripe, slab,
                  interpret=False):
  @functools.partial(
      pl.kernel,
      out_type=jax.ShapeDtypeStruct((2, NP, PW), _f32),
      mesh=_mesh(),
      compiler_params=pltpu.CompilerParams(use_tc_tiling_on_sc=False, needs_layout_passes=False),
      scratch_types=[
          pltpu.VMEM((3, CHE), jnp.int32),
          pltpu.VMEM((CHE, HC), _f32), pltpu.VMEM((CHE, HC), _f32),
          pltpu.VMEM((CHE, PW), _f32),
          pltpu.VMEM((2, 16), _f32),
          pltpu.VMEM_SHARED((NP, PW), _f32),
          pltpu.SemaphoreType.DMA, pltpu.SemaphoreType.DMA,
          pltpu.SemaphoreType.DMA,
      ],
      interpret=interpret)
  def k(sd_h, eprojs_h, xl_h, xr_h, att_h, z_h, acc_h,
        sdb, xlb, xrb, pay, attv, acc, sem1, sem2, sem3):
    cid = lax.axis_index("c")
    sid = lax.axis_index("s")
    pltpu.sync_copy(z_h, acc.at[pl.ds(sid * STR, STR)])
    pltpu.sync_copy(att_h, attv)
    plsc.subcore_barrier()
    wid = cid * NS + sid
    tbase = wid * EPT
    cbase = wid * CPT_E

    def chunk(i, carry):
      base = tbase + i * CHE
      pltpu.sync_copy(sd_h.at[cbase + i], sdb)
      g1 = pltpu.async_copy(xl_h.at[sdb.at[0]], xlb, sem1)
      g2 = pltpu.async_copy(xr_h.at[sdb.at[1]], xrb, sem2)
      g3 = pltpu.async_copy(eprojs_h.at[slab, pl.ds(base, CHE)], pay, sem3)
      g1.wait()
      g2.wait()
      g3.wait()
      att0 = attv[0]
      att1 = attv[1]
      lane = lax.broadcasted_iota(jnp.int32, (16,), 0)

      def edge4(g, carry2):
        for u in range(4):
          e = g * 4 + u
          xl0 = xlb[e, pl.ds(0, 16)]
          xl1v = xlb[e, pl.ds(16, 16)]
          m0 = xl0 + xrb[e, pl.ds(0, 16)] + pay[e, pl.ds(0, 16)]
          m1 = xl1v + xrb[e, pl.ds(16, 16)] + pay[e, pl.ds(16, 16)]
          s0 = jnp.sum(_leaky(m0) * att0)
          s1 = jnp.sum(_leaky(m1) * att1)
          p0 = jnp.exp(lax.broadcast_in_dim(s0, (16,), ()))
          p1 = jnp.exp(lax.broadcast_in_dim(s1, (16,), ()))
          pay[e, pl.ds(24, 16)] = jnp.where(lane == 8, p0, p1)
          pay[e, pl.ds(0, 16)] = p0 * xl0
          pay[e, pl.ds(16, 16)] = p1 * xl1v
        return carry2

      lax.fori_loop(0, CHE // 4, edge4, 0)
      pltpu.sync_copy(pay, acc.at[sdb.at[2]], add=True)
      return carry

    lax.fori_loop(0, CPT_E, chunk, 0)
    plsc.subcore_barrier()

    @pl.when(cid == 0)
    def _():
      pltpu.sync_copy(acc.at[pl.ds(sid * STR, STR)],
                      acc_h.at[0, pl.ds(sid * STR, STR)])

    @pl.when(cid == 1)
    def _():
      pltpu.sync_copy(acc.at[pl.ds(sid * STR, STR)],
                      acc_h.at[1, pl.ds(sid * STR, STR)])

  return k(sd, eprojs, xl, xr, att, zstripe)


def _tc_edge_dense(ps4, pd4, We1, We2, interpret=False):
  def body(ps_ref, pd_ref, w1_ref, w2_ref, out_ref):
    df = ps_ref[...] - pd_ref[...]
    d2 = jnp.sum(df * df, axis=1, keepdims=True)
    d = jnp.sqrt(d2 + 1e-12)
    offs = lax.broadcasted_iota(jnp.int32, (1, NG), 1).astype(_f32) * (
        5.0 / (NG - 1))
    ea = jnp.exp(COEFF * (d - offs) ** 2)
    e1 = jnp.dot(ea, w1_ref[...], precision=_HIGH)
    e2 = jnp.dot(ea, w2_ref[...], precision=_HIGH)
    out_ref[0, :, 0:HC] = e1
    out_ref[0, :, HC:HC + 1] = jnp.ones((BLKE, 1), _f32)
    out_ref[0, :, HC + 1:PW] = jnp.zeros((BLKE, PW - HC - 1), _f32)
    out_ref[1, :, 0:HC] = e2
    out_ref[1, :, HC:PW] = jnp.zeros((BLKE, PW - HC), _f32)

  grid = E_pad // BLKE
  return pl.pallas_call(
      body,
      grid=(grid,),
      in_specs=[
          pl.BlockSpec((BLKE, 16), lambda i: (i, 0)),
          pl.BlockSpec((BLKE, 16), lambda i: (i, 0)),
          pl.BlockSpec((NG, HC), lambda i: (0, 0)),
          pl.BlockSpec((NG, HC), lambda i: (0, 0)),
      ],
      out_specs=pl.BlockSpec((2, BLKE, PW), lambda i: (0, i, 0)),
      out_shape=jax.ShapeDtypeStruct((2, E_pad, PW), _f32),
      interpret=interpret,
  )(ps4, pd4, We1, We2)


def _tc_node_pre(pos, accS, W_init, b_init, Wl1, bl1, Wr1, br1, att1,
                 interpret=False):
  def body(pos_ref, a_ref, wi_ref, bi_ref, wl_ref, bl_ref, wr_ref, br_ref,
           at_ref, xl_o, xr_o, pl_o, le2_o, cf_o):
    x0 = jax.nn.softplus(
        jnp.dot(pos_ref[...], wi_ref[...], precision=_HIGH) + bi_ref[...])
    a0 = a_ref[0]
    a1 = a_ref[1]
    cnt = a0[:, HC:HC + 1]
    ccl = jnp.maximum(cnt, 1.0)
    le1 = a0[:, 0:HC] / ccl
    le2 = a1[:, 0:HC] / ccl
    cf_o[...] = cnt + 1.0
    le2_o[...] = le2
    xl = jnp.dot(x0, wl_ref[...], precision=_HIGH) + bl_ref[...]
    xr = jnp.dot(x0, wr_ref[...], precision=_HIGH) + br_ref[...]
    xl_o[...] = xl
    xr_o[...] = xr
    ml = _leaky(xl + xr + le1).reshape(BLKN, H, C)
    al = jnp.sum(ml * at_ref[...], axis=-1)
    pl_o[...] = jnp.exp(al)

  grid = N // BLKN
  return pl.pallas_call(
      body,
      grid=(grid,),
      in_specs=[
          pl.BlockSpec((BLKN, 3), lambda i: (i, 0)),
          pl.BlockSpec((2, BLKN, PW), lambda i: (0, i, 0)),
          pl.BlockSpec((3, C), lambda i: (0, 0)),
          pl.BlockSpec((1, C), lambda i: (0, 0)),
          pl.BlockSpec((C, HC), lambda i: (0, 0)),
          pl.BlockSpec((1, HC), lambda i: (0, 0)),
          pl.BlockSpec((C, HC), lambda i: (0, 0)),
          pl.BlockSpec((1, HC), lambda i: (0, 0)),
          pl.BlockSpec((1, H, C), lambda i: (0, 0, 0)),
      ],
      out_specs=[
          pl.BlockSpec((BLKN, HC), lambda i: (i, 0)),
          pl.BlockSpec((BLKN, HC), lambda i: (i, 0)),
          pl.BlockSpec((BLKN, H), lambda i: (i, 0)),
          pl.BlockSpec((BLKN, HC), lambda i: (i, 0)),
          pl.BlockSpec((BLKN, 1), lambda i: (i, 0)),
      ],
      out_shape=[
          jax.ShapeDtypeStruct((N, HC), _f32),
          jax.ShapeDtypeStruct((N, HC), _f32),
          jax.ShapeDtypeStruct((N, H), _f32),
          jax.ShapeDtypeStruct((N, HC), _f32),
          jax.ShapeDtypeStruct((N, 1), _f32),
      ],
      interpret=interpret,
  )(pos, accS, W_init, b_init, Wl1, bl1, Wr1, br1, att1)


def _combine(accE, xl, ploop, cntf, bias):
  asum = accE[0] + accE[1]
  num = (asum[:, 0:HC].reshape(BLKN, H, C)
         + ploop[:, :, None] * xl.reshape(BLKN, H, C))
  den = asum[:, HC:HC + H] + ploop
  out = jnp.sum(num / den[:, :, None], axis=1) * (0.5 / cntf) + bias
  return jax.nn.softplus(out)


def _tc_combine_mid(accE, xl, ploop, cntf, bias, le_n, Wl_n, bl_n, Wr_n, br_n,
                    att_n, interpret=False):
  def body(a_ref, xl_ref, pl_ref, cf_ref, b_ref, le_ref, wl_ref, bl_ref,
           wr_ref, br_ref, at_ref, xl_o, xr_o, pl_o):
    xn = _combine(a_ref[...], xl_ref[...], pl_ref[...], cf_ref[...], b_ref[...])
    xln = jnp.dot(xn, wl_ref[...], precision=_HIGH) + bl_ref[...]
    xrn = jnp.dot(xn, wr_ref[...], precision=_HIGH) + br_ref[...]
    xl_o[...] = xln
    xr_o[...] = xrn
    ml = _leaky(xln + xrn + le_ref[...]).reshape(BLKN, H, C)
    pl_o[...] = jnp.exp(jnp.sum(ml * at_ref[...], axis=-1))

  grid = N // BLKN
  return pl.pallas_call(
      body,
      grid=(grid,),
      in_specs=[
          pl.BlockSpec((2, BLKN, PW), lambda i: (0, i, 0)),
          pl.BlockSpec((BLKN, HC), lambda i: (i, 0)),
          pl.BlockSpec((BLKN, H), lambda i: (i, 0)),
          pl.BlockSpec((BLKN, 1), lambda i: (i, 0)),
          pl.BlockSpec((1, C), lambda i: (0, 0)),
          pl.BlockSpec((BLKN, HC), lambda i: (i, 0)),
          pl.BlockSpec((C, HC), lambda i: (0, 0)),
          pl.BlockSpec((1, HC), lambda i: (0, 0)),
          pl.BlockSpec((C, HC), lambda i: (0, 0)),
          pl.BlockSpec((1, HC), lambda i: (0, 0)),
          pl.BlockSpec((1, H, C), lambda i: (0, 0, 0)),
      ],
      out_specs=[
          pl.BlockSpec((BLKN, HC), lambda i: (i, 0)),
          pl.BlockSpec((BLKN, HC), lambda i: (i, 0)),
          pl.BlockSpec((BLKN, H), lambda i: (i, 0)),
      ],
      out_shape=[
          jax.ShapeDtypeStruct((N, HC), _f32),
          jax.ShapeDtypeStruct((N, HC), _f32),
          jax.ShapeDtypeStruct((N, H), _f32),
      ],
      interpret=interpret,
  )(accE, xl, ploop, cntf, bias, le_n, Wl_n, bl_n, Wr_n, br_n, att_n)


def _tc_combine_final(accE, xl, ploop, cntf, bias, W_p1, b_p1, W_p2, b_p2,
                      sig, interpret=False):
  def body(a_ref, xl_ref, pl_ref, cf_ref, b_ref, w1_ref, b1_ref, w2_ref,
           b2_ref, sg_ref, out_o):
    xn = _combine(a_ref[...], xl_ref[...], pl_ref[...], cf_ref[...], b_ref[...])
    y = jax.nn.softplus(
        jnp.dot(xn, w1_ref[...], precision=_HIGH) + b1_ref[...])
    sc = jnp.dot(y, w2_ref[...], precision=_HIGH) + b2_ref[...]
    out_o[...] = sc / sg_ref[...]

  grid = N // BLKN
  return pl.pallas_call(
      body,
      grid=(grid,),
      in_specs=[
          pl.BlockSpec((2, BLKN, PW), lambda i: (0, i, 0)),
          pl.BlockSpec((BLKN, HC), lambda i: (i, 0)),
          pl.BlockSpec((BLKN, H), lambda i: (i, 0)),
          pl.BlockSpec((BLKN, 1), lambda i: (i, 0)),
          pl.BlockSpec((1, C), lambda i: (0, 0)),
          pl.BlockSpec((C, C), lambda i: (0, 0)),
          pl.BlockSpec((1, C), lambda i: (0, 0)),
          pl.BlockSpec((C, 3), lambda i: (0, 0)),
          pl.BlockSpec((1, 3), lambda i: (0, 0)),
          pl.BlockSpec((BLKN, 1), lambda i: (i, 0)),
      ],
      out_specs=pl.BlockSpec((BLKN, 3), lambda i: (i, 0)),
      out_shape=jax.ShapeDtypeStruct((N, 3), _f32),
      interpret=interpret,
  )(accE, xl, ploop, cntf, bias, W_p1, b_p1, W_p2, b_p2, sig)


def kernel(pos, edge_index, sigmas, W_init, b_init, Wl1, bl1, Wr1, br1, We1,
           att1, bias1, Wl2, bl2, Wr2, br2, We2, att2, bias2, W_p1, b_p1,
           W_p2, b_p2):
  src = edge_index[0]
  dst = edge_index[1]
  padn = E_pad - E
  srcp = jnp.concatenate([src, jnp.zeros((padn,), jnp.int32)])
  dstg = jnp.concatenate([dst, jnp.zeros((padn,), jnp.int32)])
  dsts = jnp.concatenate([dst, jnp.full((padn,), N, jnp.int32)])
  pos4 = jnp.pad(pos, ((0, 0), (0, 13)))
  zstripe = jnp.zeros((STR, PW), _f32)
  sig = sigmas.reshape(N, 1)
  b_init2 = b_init.reshape(1, C)
  bl1r, br1r = bl1.reshape(1, HC), br1.reshape(1, HC)
  bl2r, br2r = bl2.reshape(1, HC), br2.reshape(1, HC)
  bias1r, bias2r = bias1.reshape(1, C), bias2.reshape(1, C)
  att1r = att1.reshape(H, C)
  att2r = att2.reshape(H, C)

  sd = jnp.stack([srcp.reshape(-1, CHE), dstg.reshape(-1, CHE),
                  dsts.reshape(-1, CHE)], axis=1)
  ps4, pd4 = _sc_gather_pos(pos4, srcp, dstg)
  eprojs = _tc_edge_dense(ps4, pd4, We1, We2)
  accS = _sc_scatter_pre(eprojs, dsts, zstripe)
  xl, xr, ploop, le2, cntf = _tc_node_pre(
      pos, accS, W_init, b_init2, Wl1, bl1r, Wr1, br1r, att1)

  accE = _sc_edge_pass(sd, eprojs, xl, xr, att1r, zstripe, 0)
  xl, xr, ploop2 = _tc_combine_mid(
      accE, xl, ploop, cntf, bias1r, le2, Wl2, bl2r, Wr2, br2r, att2)
  accE = _sc_edge_pass(sd, eprojs, xl, xr, att2r, zstripe, 1)
  xl, xr, ploop3 = _tc_combine_mid(
      accE, xl, ploop2, cntf, bias2r, le2, Wl2, bl2r, Wr2, br2r, att2)
  accE = _sc_edge_pass(sd, eprojs, xl, xr, att2r, zstripe, 1)
  scores = _tc_combine_final(
      accE, xl, ploop3, cntf, bias2r, W_p1, b_p1.reshape(1, C), W_p2,
      b_p2.reshape(1, 3), sig)
  return scores
